# baseline (speedup 1.0000x reference)
import functools
import math

import jax
import jax.numpy as jnp
from jax import lax
from jax.experimental import pallas as pl
from jax.experimental.pallas import tpu as pltpu
from jax.experimental.pallas import tpu_sc as plsc

D = 64
SCALE = math.sqrt(D)
NC, NS = 2, 16
NW = NC * NS
IB = 128


@functools.lru_cache(maxsize=None)
def _emb_kernel(B):
    rows_per_w = B // (NW * IB)
    mesh = plsc.VectorSubcoreMesh(
        core_axis_name="c", subcore_axis_name="s",
        num_cores=NC, num_subcores=NS)

    @functools.partial(
        pl.kernel,
        out_type=jax.ShapeDtypeStruct((B, D), jnp.float32),
        mesh=mesh,
        scratch_types=[
            pltpu.VMEM((rows_per_w, IB), jnp.int32),
            pltpu.VMEM((IB, D), jnp.float32),
            pltpu.SemaphoreType.DMA,
        ],
        compiler_params=pltpu.CompilerParams(use_tc_tiling_on_sc=False),
    )
    def body(x_hbm, lut_hbm, out_hbm, idx_v, rows_v, sem):
        wid = lax.axis_index("s") * NC + lax.axis_index("c")
        base_row = wid * rows_per_w
        pltpu.sync_copy(x_hbm.at[pl.ds(base_row, rows_per_w)], idx_v)

        def step(j, carry):
            pltpu.async_copy(lut_hbm.at[idx_v.at[j]], rows_v, sem).wait()

            def scale_row(r, c):
                for k in range(D // 16):
                    sl = pl.ds(k * 16, 16)
                    rows_v[r, sl] = rows_v[r, sl] * SCALE
                return c

            lax.fori_loop(0, IB, scale_row, 0)
            pltpu.sync_copy(rows_v, out_hbm.at[pl.ds((base_row + j) * IB, IB)])
            return carry

        lax.fori_loop(0, rows_per_w, step, 0)

    return body


@jax.jit
def kernel(x, lut):
    s0, s1 = x.shape
    B = s0 * s1
    x2 = x.reshape(B // IB, IB).astype(jnp.int32)
    out = _emb_kernel(B)(x2, lut)
    return out.reshape(s0, s1, D)


# speedup vs baseline: 1.2025x; 1.2025x over previous
import functools
import math

import jax
import jax.numpy as jnp
from jax import lax
from jax.experimental import pallas as pl
from jax.experimental.pallas import tpu as pltpu
from jax.experimental.pallas import tpu_sc as plsc

D = 64
SCALE = math.sqrt(D)
NC, NS = 2, 16
NW = NC * NS
IB = 128
K = 4


@functools.lru_cache(maxsize=None)
def _emb_kernel(B):
    rows_per_w = B // (NW * IB)
    ngroups = rows_per_w // K
    assert ngroups % 2 == 0
    npairs = ngroups // 2
    mesh = plsc.VectorSubcoreMesh(
        core_axis_name="c", subcore_axis_name="s",
        num_cores=NC, num_subcores=NS)

    @functools.partial(
        pl.kernel,
        out_type=jax.ShapeDtypeStruct((B, D), jnp.float32),
        mesh=mesh,
        scratch_types=[
            pltpu.VMEM((rows_per_w, IB), jnp.int32),
            pltpu.VMEM((K * IB, D), jnp.float32),
            pltpu.VMEM((K * IB, D), jnp.float32),
            pltpu.SemaphoreType.DMA,
            pltpu.SemaphoreType.DMA,
            pltpu.SemaphoreType.DMA,
            pltpu.SemaphoreType.DMA,
        ],
        compiler_params=pltpu.CompilerParams(use_tc_tiling_on_sc=False),
    )
    def body(x_hbm, lut_hbm, out_hbm, idx_v, rows_a, rows_b,
             gsem_a, gsem_b, ssem_a, ssem_b):
        wid = lax.axis_index("s") * NC + lax.axis_index("c")
        base_row = wid * rows_per_w
        out_base = base_row * IB
        pltpu.sync_copy(x_hbm.at[pl.ds(base_row, rows_per_w)], idx_v)

        def start_gather(g, rows_v, gsem):
            for b in range(K):
                pltpu.async_copy(
                    lut_hbm.at[idx_v.at[g * K + b]],
                    rows_v.at[pl.ds(b * IB, IB)], gsem)

        def drain_gather(rows_v, gsem):
            for b in range(K):
                pltpu.make_async_copy(
                    lut_hbm.at[idx_v.at[b]],
                    rows_v.at[pl.ds(b * IB, IB)], gsem).wait()

        def scale(rows_v):
            def row(r, c):
                for b in range(K):
                    for k in range(D // 16):
                        sl = pl.ds(k * 16, 16)
                        rows_v[b * IB + r, sl] = rows_v[b * IB + r, sl] * SCALE
                return c
            lax.fori_loop(0, IB, row, 0)

        def start_scatter(g, rows_v, ssem):
            pltpu.async_copy(
                rows_v, out_hbm.at[pl.ds(out_base + g * (K * IB), K * IB)],
                ssem)

        def drain_scatter(g, rows_v, ssem):
            pltpu.make_async_copy(
                rows_v, out_hbm.at[pl.ds(out_base + g * (K * IB), K * IB)],
                ssem).wait()

        start_gather(0, rows_a, gsem_a)
        start_gather(1, rows_b, gsem_b)

        def pair(i2, c):
            ga = 2 * i2
            drain_gather(rows_a, gsem_a)
            scale(rows_a)
            start_scatter(ga, rows_a, ssem_a)
            drain_gather(rows_b, gsem_b)
            scale(rows_b)
            start_scatter(ga + 1, rows_b, ssem_b)
            drain_scatter(ga, rows_a, ssem_a)
            start_gather(ga + 2, rows_a, gsem_a)
            drain_scatter(ga + 1, rows_b, ssem_b)
            start_gather(ga + 3, rows_b, gsem_b)
            return c

        lax.fori_loop(0, npairs - 1, pair, 0)

        gl = ngroups - 2
        drain_gather(rows_a, gsem_a)
        scale(rows_a)
        start_scatter(gl, rows_a, ssem_a)
        drain_gather(rows_b, gsem_b)
        scale(rows_b)
        start_scatter(gl + 1, rows_b, ssem_b)
        drain_scatter(gl, rows_a, ssem_a)
        drain_scatter(gl + 1, rows_b, ssem_b)

    return body


@jax.jit
def kernel(x, lut):
    s0, s1 = x.shape
    B = s0 * s1
    x2 = x.reshape(B // IB, IB).astype(jnp.int32)
    out = _emb_kernel(B)(x2, lut)
    return out.reshape(s0, s1, D)
